# baseline (speedup 1.0000x reference)
import functools

import jax
import jax.numpy as jnp
from jax import lax
from jax.experimental import pallas as pl
from jax.experimental.pallas import tpu as pltpu
from jax.experimental.pallas import tpu_sc as plsc

F = 26
V = 100000
D = 16
B = 16384

NC = 2
NS = 16
NW = NC * NS

ROWS = F * D
PER_W = ROWS // NW

SEG = 33408
OFFS = (0, SEG, 2 * SEG)
SIZES = (SEG, SEG, 33152)
VT = 99968
TW = V - VT
NSEG = 3

OCH = 4096
NOCH = B // OCH


def _body(tab_hbm, tail_hbm, idx_hbm, out_hbm, buf0, buf1, ob, idx_v,
          tail_v, semA, semB, tsem, wsem):
    wid = lax.axis_index("s") * NC + lax.axis_index("c")
    r0 = wid * PER_W

    bufs = (buf0, buf1)
    sems = (semA, semB)
    pend = [None, None]
    wpend = []

    lanes = lax.iota(jnp.int32, 16)

    def issue(g):
        j, k = divmod(g, NSEG)
        b = g % 2
        pend[b] = pltpu.async_copy(
            tab_hbm.at[r0 + j, pl.ds(OFFS[k], SIZES[k])],
            bufs[b].at[pl.ds(0, SIZES[k])],
            sems[b],
        )

    issue(0)
    tpend = [
        pltpu.async_copy(tail_hbm.at[r0 + jj], tail_v.at[jj], tsem)
        for jj in range(PER_W)
    ]

    for j in range(PER_W):
        r = r0 + j
        f = r // D

        if j == 0:
            pltpu.sync_copy(idx_hbm.at[f], idx_v)
        else:
            f_prev = (r - 1) // D

            @pl.when(f != f_prev)
            def _():
                pltpu.sync_copy(idx_hbm.at[f], idx_v)

        for k in range(NSEG):
            g = NSEG * j + k
            buf = bufs[g % 2]
            pend[g % 2].wait()
            if g + 1 < NSEG * PER_W:
                issue(g + 1)

            if k == 0:
                for p in wpend:
                    p.wait()
                wpend = []

                @plsc.parallel_loop(0, B // 16, unroll=8)
                def p0(gi):
                    iv = idx_v[pl.ds(gi * 16, 16)]
                    loc = jnp.minimum(iv, SIZES[0] - 1)
                    ob[pl.ds(gi * 16, 16)] = plsc.load_gather(buf, [loc])

            elif k == 1:

                @plsc.parallel_loop(0, B // 16, unroll=8)
                def p1(gi):
                    iv = idx_v[pl.ds(gi * 16, 16)]
                    loc = jnp.clip(iv - OFFS[1], 0, SIZES[1] - 1)
                    gv = plsc.load_gather(buf, [loc])
                    pos = gi * 16 + lanes
                    plsc.store_scatter(ob, [pos], gv, mask=iv >= OFFS[1])

            else:
                if tpend is not None:
                    for tp in tpend:
                        tp.wait()
                    tpend = None
                jrow = jnp.full((16,), j, jnp.int32)

                @plsc.parallel_loop(0, B // 16, unroll=8)
                def p2(gi, jrow=jrow):
                    iv = idx_v[pl.ds(gi * 16, 16)]
                    loc = jnp.clip(iv - OFFS[2], 0, SIZES[2] - 1)
                    gv = plsc.load_gather(buf, [loc])
                    pos = gi * 16 + lanes
                    plsc.store_scatter(ob, [pos], gv, mask=iv >= OFFS[2])
                    loct = jnp.clip(iv - VT, 0, TW - 1)
                    gt = plsc.load_gather(tail_v, [jrow, loct])
                    plsc.store_scatter(ob, [pos], gt, mask=iv >= VT)

                for c in range(NOCH):
                    wpend.append(pltpu.async_copy(
                        ob.at[pl.ds(c * OCH, OCH)],
                        out_hbm.at[r, pl.ds(c * OCH, OCH)],
                        wsem,
                    ))

    for p in wpend:
        p.wait()


@jax.jit
def _run(tab_t, tail_t, idx_t):
    kern = functools.partial(
        pl.kernel,
        mesh=plsc.VectorSubcoreMesh(core_axis_name="c", subcore_axis_name="s"),
        out_type=jax.ShapeDtypeStruct((ROWS, B), jnp.float32),
        scratch_types=[
            pltpu.VMEM((SEG,), jnp.float32),
            pltpu.VMEM((SEG,), jnp.float32),
            pltpu.VMEM((B,), jnp.float32),
            pltpu.VMEM((B,), jnp.int32),
            pltpu.VMEM((PER_W, TW), jnp.float32),
            pltpu.SemaphoreType.DMA,
            pltpu.SemaphoreType.DMA,
            pltpu.SemaphoreType.DMA,
            pltpu.SemaphoreType.DMA,
        ],
        compiler_params=pltpu.CompilerParams(
            use_tc_tiling_on_sc=True, needs_layout_passes=False
        ),
    )(_body)
    return kern(tab_t, tail_t, idx_t)


def kernel(indices, tables):
    tab_t = tables.transpose(0, 2, 1).reshape(ROWS, V)
    tail_t = tables[:, VT:, :].transpose(0, 2, 1).reshape(ROWS, TW)
    idx_t = indices.T.astype(jnp.int32)
    out_t = _run(tab_t, tail_t, idx_t)
    return out_t.T


# speedup vs baseline: 1.0839x; 1.0839x over previous
import functools

import jax
import jax.numpy as jnp
from jax import lax
from jax.experimental import pallas as pl
from jax.experimental.pallas import tpu as pltpu
from jax.experimental.pallas import tpu_sc as plsc

F = 26
V = 100000
D = 16
B = 16384

NC = 2
NS = 16
NW = NC * NS

ROWS = F * D
PER_W = ROWS // NW

SA = 66816
SB = 33152
VT = 99968
TW = V - VT

OCH = 4096
NOCH = B // OCH
NPAR = 3


def _body(tab_hbm, tail_hbm, idx_hbm, out_hbm, bufA, bufB, ob0, ob1, ob2,
          idx_v, tail_v, semA, semB, tsem, wsem):
    wid = lax.axis_index("s") * NC + lax.axis_index("c")
    r0 = wid * PER_W

    obufs = (ob0, ob1, ob2)
    lanes = lax.iota(jnp.int32, 16)

    pendA = [None]
    pendB = [None]
    wpend = [None, None, None]

    def issueA(j):
        pendA[0] = pltpu.async_copy(
            tab_hbm.at[r0 + j, pl.ds(0, SA)], bufA, semA
        )

    def issueB(j):
        pendB[0] = pltpu.async_copy(
            tab_hbm.at[r0 + j, pl.ds(SA, SB)], bufB, semB
        )

    issueA(0)
    issueB(0)
    tpend = [
        pltpu.async_copy(tail_hbm.at[r0 + jj], tail_v.at[jj], tsem)
        for jj in range(PER_W)
    ]

    for j in range(PER_W):
        r = r0 + j
        f = r // D

        if j == 0:
            pltpu.sync_copy(idx_hbm.at[f], idx_v)
        else:
            f_prev = (r - 1) // D

            @pl.when(f != f_prev)
            def _():
                pltpu.sync_copy(idx_hbm.at[f], idx_v)

        def passA(q):
            par = q % NPAR
            if wpend[par] is not None:
                wpend[par].wait()
                wpend[par] = None
            ob = obufs[par]

            @plsc.parallel_loop(0, OCH // 16, unroll=4)
            def pA(gi, q=q, ob=ob):
                iv = idx_v[pl.ds(q * OCH + gi * 16, 16)]
                loc = jnp.minimum(iv, SA - 1)
                ob[pl.ds(gi * 16, 16)] = plsc.load_gather(bufA, [loc])

        def passB(q, jrow):
            par = q % NPAR
            ob = obufs[par]

            @plsc.parallel_loop(0, OCH // 16, unroll=4)
            def pB(gi, q=q, ob=ob, jrow=jrow):
                iv = idx_v[pl.ds(q * OCH + gi * 16, 16)]
                loc = jnp.clip(iv - SA, 0, SB - 1)
                gv = plsc.load_gather(bufB, [loc])
                pos = gi * 16 + lanes
                plsc.store_scatter(ob, [pos], gv, mask=iv >= SA)
                loct = jnp.clip(iv - VT, 0, TW - 1)
                gt = plsc.load_gather(tail_v, [jrow, loct])
                plsc.store_scatter(ob, [pos], gt, mask=iv >= VT)

            wpend[par] = pltpu.async_copy(
                ob, out_hbm.at[r, pl.ds(q * OCH, OCH)], wsem
            )

        pendA[0].wait()
        passA(0)
        passA(1)
        passA(2)

        pendB[0].wait()
        if tpend is not None:
            for tp in tpend:
                tp.wait()
            tpend = None
        jrow = jnp.full((16,), j, jnp.int32)

        passB(0, jrow)
        passA(3)
        if j + 1 < PER_W:
            issueA(j + 1)
        passB(1, jrow)
        passB(2, jrow)
        passB(3, jrow)
        if j + 1 < PER_W:
            issueB(j + 1)

    for p in wpend:
        if p is not None:
            p.wait()


@jax.jit
def _run(tab_t, tail_t, idx_t):
    kern = functools.partial(
        pl.kernel,
        mesh=plsc.VectorSubcoreMesh(core_axis_name="c", subcore_axis_name="s"),
        out_type=jax.ShapeDtypeStruct((ROWS, B), jnp.float32),
        scratch_types=[
            pltpu.VMEM((SA,), jnp.float32),
            pltpu.VMEM((SB,), jnp.float32),
            pltpu.VMEM((OCH,), jnp.float32),
            pltpu.VMEM((OCH,), jnp.float32),
            pltpu.VMEM((OCH,), jnp.float32),
            pltpu.VMEM((B,), jnp.int32),
            pltpu.VMEM((PER_W, TW), jnp.float32),
            pltpu.SemaphoreType.DMA,
            pltpu.SemaphoreType.DMA,
            pltpu.SemaphoreType.DMA,
            pltpu.SemaphoreType.DMA,
        ],
        compiler_params=pltpu.CompilerParams(
            use_tc_tiling_on_sc=True, needs_layout_passes=False
        ),
    )(_body)
    return kern(tab_t, tail_t, idx_t)


def kernel(indices, tables):
    tab_t = tables.transpose(0, 2, 1).reshape(ROWS, V)
    tail_t = tables[:, VT:, :].transpose(0, 2, 1).reshape(ROWS, TW)
    idx_t = indices.T.astype(jnp.int32)
    out_t = _run(tab_t, tail_t, idx_t)
    return out_t.T


# speedup vs baseline: 1.1885x; 1.0965x over previous
import functools

import jax
import jax.numpy as jnp
from jax import lax
from jax.experimental import pallas as pl
from jax.experimental.pallas import tpu as pltpu
from jax.experimental.pallas import tpu_sc as plsc

F = 26
V = 100000
D = 16
B = 16384

NC = 2
NS = 16
NW = NC * NS

ROWS = F * D
PER_W = ROWS // NW

SEG = 33408
OFFS = (0, SEG, 2 * SEG)
SIZES = (SEG, SEG, 33152)
VT = 99968
TW = V - VT

OCH = 4096
NOCH = B // OCH


def _body(tab_hbm, tail_hbm, idx_hbm, out_hbm, row_v, idx_v, tail_v, ob0,
          ob1, sem0, sem1, sem2, semt, wsem0, wsem1):
    wid = lax.axis_index("s") * NC + lax.axis_index("c")
    r0 = wid * PER_W

    obufs = (ob0, ob1)
    wsems = (wsem0, wsem1)
    rsems = (sem0, sem1, sem2)
    pending = [None, None]

    def stage_row(j):
        r = r0 + j
        return [
            pltpu.async_copy(
                tab_hbm.at[r, pl.ds(OFFS[k], SIZES[k])],
                row_v.at[pl.ds(OFFS[k], SIZES[k])],
                rsems[k],
            )
            for k in range(3)
        ]

    tpend = [
        pltpu.async_copy(tail_hbm.at[r0 + jj], tail_v.at[jj], semt)
        for jj in range(PER_W)
    ]

    for j in range(PER_W):
        r = r0 + j
        f = r // D

        rcps = stage_row(j)
        if tpend is not None:
            for tp in tpend:
                tp.wait()
            tpend = None
        row_v[pl.ds(VT, 16)] = tail_v[j, pl.ds(0, 16)]
        row_v[pl.ds(VT + 16, 16)] = tail_v[j, pl.ds(16, 16)]

        if j == 0:
            pltpu.sync_copy(idx_hbm.at[f], idx_v)
        else:
            f_prev = (r - 1) // D

            @pl.when(f != f_prev)
            def _():
                pltpu.sync_copy(idx_hbm.at[f], idx_v)

        for cp in rcps:
            cp.wait()

        for c in range(NOCH):
            k = c % 2
            if pending[k] is not None:
                pending[k].wait()
            ob = obufs[k]

            @plsc.parallel_loop(0, OCH // 16, unroll=8)
            def gather(g, c=c, ob=ob):
                iv = idx_v[pl.ds(c * OCH + g * 16, 16)]
                ob[pl.ds(g * 16, 16)] = plsc.load_gather(row_v, [iv])

            pending[k] = pltpu.async_copy(
                ob, out_hbm.at[r, pl.ds(c * OCH, OCH)], wsems[k]
            )

    for p in pending:
        if p is not None:
            p.wait()


@jax.jit
def _run(tab_t, tail_t, idx_t):
    kern = functools.partial(
        pl.kernel,
        mesh=plsc.VectorSubcoreMesh(core_axis_name="c", subcore_axis_name="s"),
        out_type=jax.ShapeDtypeStruct((ROWS, B), jnp.float32),
        scratch_types=[
            pltpu.VMEM((V,), jnp.float32),
            pltpu.VMEM((B,), jnp.int32),
            pltpu.VMEM((PER_W, TW), jnp.float32),
            pltpu.VMEM((OCH,), jnp.float32),
            pltpu.VMEM((OCH,), jnp.float32),
            pltpu.SemaphoreType.DMA,
            pltpu.SemaphoreType.DMA,
            pltpu.SemaphoreType.DMA,
            pltpu.SemaphoreType.DMA,
            pltpu.SemaphoreType.DMA,
            pltpu.SemaphoreType.DMA,
        ],
        compiler_params=pltpu.CompilerParams(
            use_tc_tiling_on_sc=True, needs_layout_passes=False
        ),
    )(_body)
    return kern(tab_t, tail_t, idx_t)


def kernel(indices, tables):
    tab_t = tables.transpose(0, 2, 1).reshape(ROWS, V)
    tail_t = tables[:, VT:, :].transpose(0, 2, 1).reshape(ROWS, TW)
    idx_t = indices.T.astype(jnp.int32)
    out_t = _run(tab_t, tail_t, idx_t)
    return out_t.T


# speedup vs baseline: 1.1894x; 1.0007x over previous
import functools

import jax
import jax.numpy as jnp
from jax import lax
from jax.experimental import pallas as pl
from jax.experimental.pallas import tpu as pltpu
from jax.experimental.pallas import tpu_sc as plsc

F = 26
V = 100000
D = 16
B = 16384

NC = 2
NS = 16
NW = NC * NS

ROWS = F * D
PER_W = ROWS // NW
OCH = 4096
NOCH = B // OCH


def _body(tab_hbm, idx_hbm, out_hbm, row_v, idx_v, ob0, ob1, sem0, sem1):
    wid = lax.axis_index("s") * NC + lax.axis_index("c")
    r0 = wid * PER_W

    obufs = (ob0, ob1)
    sems = (sem0, sem1)
    pending = [None, None]

    for j in range(PER_W):
        r = r0 + j
        f = r // D

        if j == 0:
            pltpu.sync_copy(idx_hbm.at[f], idx_v)
        else:
            f_prev = (r - 1) // D

            @pl.when(f != f_prev)
            def _():
                pltpu.sync_copy(idx_hbm.at[f], idx_v)

        pltpu.sync_copy(tab_hbm.at[r], row_v)

        for c in range(NOCH):
            k = c % 2
            if pending[k] is not None:
                pending[k].wait()
            ob = obufs[k]

            @plsc.parallel_loop(0, OCH // 16, unroll=16)
            def gather(g, c=c, ob=ob):
                iv = idx_v[pl.ds(c * OCH + g * 16, 16)]
                ob[pl.ds(g * 16, 16)] = plsc.load_gather(row_v, [iv])

            pending[k] = pltpu.async_copy(
                ob, out_hbm.at[r, pl.ds(c * OCH, OCH)], sems[k]
            )

    for p in pending:
        if p is not None:
            p.wait()


@jax.jit
def _run(tab_t, idx_t):
    kern = functools.partial(
        pl.kernel,
        mesh=plsc.VectorSubcoreMesh(core_axis_name="c", subcore_axis_name="s"),
        out_type=jax.ShapeDtypeStruct((ROWS, B), jnp.float32),
        scratch_types=[
            pltpu.VMEM((V,), jnp.float32),
            pltpu.VMEM((B,), jnp.int32),
            pltpu.VMEM((OCH,), jnp.float32),
            pltpu.VMEM((OCH,), jnp.float32),
            pltpu.SemaphoreType.DMA,
            pltpu.SemaphoreType.DMA,
        ],
        compiler_params=pltpu.CompilerParams(
            use_tc_tiling_on_sc=True, needs_layout_passes=False
        ),
    )(_body)
    return kern(tab_t, idx_t)


def kernel(indices, tables):
    tab_t = tables.transpose(0, 2, 1).reshape(ROWS, V)
    idx_t = indices.T.astype(jnp.int32)
    out_t = _run(tab_t, idx_t)
    return out_t.T


# speedup vs baseline: 1.2252x; 1.0301x over previous
import functools

import jax
import jax.numpy as jnp
from jax import lax
from jax.experimental import pallas as pl
from jax.experimental.pallas import tpu as pltpu
from jax.experimental.pallas import tpu_sc as plsc

F = 26
V = 100000
D = 16
B = 16384

NC = 2
NS = 16
NW = NC * NS

ROWS = F * D
PER_W = ROWS // NW
OCH = 4096
NOCH = B // OCH


def _body(tab_hbm, idx_hbm, out_hbm, row_v, idx_v, ob0, ob1, sem0, sem1):
    wid = lax.axis_index("s") * NC + lax.axis_index("c")
    r0 = wid * PER_W

    obufs = (ob0, ob1)
    sems = (sem0, sem1)
    pending = [None, None]

    for j in range(PER_W):
        r = r0 + j
        f = r // D

        if j == 0:
            pltpu.sync_copy(idx_hbm.at[f], idx_v)
        else:
            f_prev = (r - 1) // D

            @pl.when(f != f_prev)
            def _():
                pltpu.sync_copy(idx_hbm.at[f], idx_v)

        pltpu.sync_copy(tab_hbm.at[r], row_v)

        for c in range(NOCH):
            k = c % 2
            if pending[k] is not None:
                pending[k].wait()
            ob = obufs[k]

            @plsc.parallel_loop(0, OCH // 16, unroll=8)
            def gather(g, c=c, ob=ob):
                iv = idx_v[pl.ds(c * OCH + g * 16, 16)]
                ob[pl.ds(g * 16, 16)] = plsc.load_gather(row_v, [iv])

            pending[k] = pltpu.async_copy(
                ob, out_hbm.at[r, pl.ds(c * OCH, OCH)], sems[k]
            )

    for p in pending:
        if p is not None:
            p.wait()


@jax.jit
def _run(tab_t, idx_t):
    kern = functools.partial(
        pl.kernel,
        mesh=plsc.VectorSubcoreMesh(core_axis_name="c", subcore_axis_name="s"),
        out_type=jax.ShapeDtypeStruct((ROWS, B), jnp.float32),
        scratch_types=[
            pltpu.VMEM((V,), jnp.float32),
            pltpu.VMEM((B,), jnp.int32),
            pltpu.VMEM((OCH,), jnp.float32),
            pltpu.VMEM((OCH,), jnp.float32),
            pltpu.SemaphoreType.DMA,
            pltpu.SemaphoreType.DMA,
        ],
        compiler_params=pltpu.CompilerParams(
            use_tc_tiling_on_sc=True, needs_layout_passes=False
        ),
    )(_body)
    return kern(tab_t, idx_t)


def kernel(indices, tables):
    tab_t = tables.transpose(0, 2, 1).reshape(ROWS, V)
    idx_t = indices.T.astype(jnp.int32)
    out_t = _run(tab_t, idx_t)
    return out_t.T


# speedup vs baseline: 1.2381x; 1.0106x over previous
import functools

import jax
import jax.numpy as jnp
from jax import lax
from jax.experimental import pallas as pl
from jax.experimental.pallas import tpu as pltpu
from jax.experimental.pallas import tpu_sc as plsc

F = 26
V = 100000
D = 16
B = 16384

NC = 2
NS = 16
NW = NC * NS

ROWS = F * D
PER_W = ROWS // NW
OCH = 4096
NOCH = B // OCH


def _body(tab_hbm, idx_hbm, out_hbm, row_v, idx_v, ob0, ob1, sem0, sem1):
    wid = lax.axis_index("s") * NC + lax.axis_index("c")
    r0 = wid * PER_W

    obufs = (ob0, ob1)
    sems = (sem0, sem1)
    pending = [None, None]

    phase = wid % PER_W

    for j in range(PER_W):
        r = r0 + lax.rem(j + phase, PER_W)
        f = r // D

        if j == 0:
            pltpu.sync_copy(idx_hbm.at[f], idx_v)
        else:
            r_prev = r0 + lax.rem(j - 1 + phase, PER_W)
            f_prev = r_prev // D

            @pl.when(f != f_prev)
            def _():
                pltpu.sync_copy(idx_hbm.at[f], idx_v)

        pltpu.sync_copy(tab_hbm.at[r], row_v)

        for c in range(NOCH):
            k = c % 2
            if pending[k] is not None:
                pending[k].wait()
            ob = obufs[k]

            @plsc.parallel_loop(0, OCH // 16, unroll=8)
            def gather(g, c=c, ob=ob):
                iv = idx_v[pl.ds(c * OCH + g * 16, 16)]
                ob[pl.ds(g * 16, 16)] = plsc.load_gather(row_v, [iv])

            pending[k] = pltpu.async_copy(
                ob, out_hbm.at[r, pl.ds(c * OCH, OCH)], sems[k]
            )

    for p in pending:
        if p is not None:
            p.wait()


@jax.jit
def _run(tab_t, idx_t):
    kern = functools.partial(
        pl.kernel,
        mesh=plsc.VectorSubcoreMesh(core_axis_name="c", subcore_axis_name="s"),
        out_type=jax.ShapeDtypeStruct((ROWS, B), jnp.float32),
        scratch_types=[
            pltpu.VMEM((V,), jnp.float32),
            pltpu.VMEM((B,), jnp.int32),
            pltpu.VMEM((OCH,), jnp.float32),
            pltpu.VMEM((OCH,), jnp.float32),
            pltpu.SemaphoreType.DMA,
            pltpu.SemaphoreType.DMA,
        ],
        compiler_params=pltpu.CompilerParams(
            use_tc_tiling_on_sc=True, needs_layout_passes=False
        ),
    )(_body)
    return kern(tab_t, idx_t)


def kernel(indices, tables):
    tab_t = tables.transpose(0, 2, 1).reshape(ROWS, V)
    idx_t = indices.T.astype(jnp.int32)
    out_t = _run(tab_t, idx_t)
    return out_t.T
